# baseline (speedup 1.0000x reference)
import functools

import jax
import jax.numpy as jnp
from jax import lax
from jax.experimental import pallas as pl
from jax.experimental.pallas import tpu as pltpu
from jax.experimental.pallas import tpu_sc as plsc

R, C = 64, 8192
NS, L = 16, 16
RPW = R // NS
ELEMS = RPW * C
G = 64
GELEMS = G * L
NG = ELEMS // GELEMS
U = 4
CHUNK_GROUPS = (2, 2, 4, 8, 8, 8)
INT_MAX = 2**31 - 1


def _lex_merge(av, ai, bv, bi):
    upd = (bv < av) | ((bv == av) & (bi < ai))
    return jnp.where(upd, bv, av), jnp.where(upd, bi, ai)


_sc_mesh = plsc.VectorSubcoreMesh(
    core_axis_name="c", subcore_axis_name="s", num_cores=1
)


@functools.partial(
    pl.kernel,
    out_type=jax.ShapeDtypeStruct((L,), jnp.int32),
    mesh=_sc_mesh,
    scratch_types=[
        pltpu.VMEM((ELEMS,), jnp.float32),
        pltpu.VMEM((NG * L,), jnp.float32),
        pltpu.VMEM((L,), jnp.float32),
        pltpu.VMEM((L,), jnp.int32),
        pltpu.VMEM((NS * L,), jnp.float32),
        pltpu.VMEM((NS * L,), jnp.int32),
        pltpu.VMEM_SHARED((NS * L,), jnp.float32),
        pltpu.VMEM_SHARED((NS * L,), jnp.int32),
        [pltpu.SemaphoreType.DMA] * len(CHUNK_GROUPS),
    ],
)
def _sc_argmin(
    a_hbm, out, buf, gmin_ref, vmin_ref, vidx_ref, mv_ref, mi_ref,
    sh_v, sh_i, sems
):
    s = lax.axis_index("s")
    row0 = s * RPW
    base = row0 * C

    copies = []
    goff = 0
    for k, cg in enumerate(CHUNK_GROUPS):
        e0 = goff * GELEMS
        n = cg * GELEMS
        copies.append(
            pltpu.async_copy(
                a_hbm.at[row0 + e0 // C, pl.ds(e0 % C, n)],
                buf.at[pl.ds(e0, n)],
                sems[k],
            )
        )
        goff += cg

    lane = lax.iota(jnp.int32, L)
    inf = jnp.float32(jnp.inf)

    gm = jnp.full((L,), inf, jnp.float32)
    goff = 0
    for k, cg in enumerate(CHUNK_GROUPS):
        copies[k].wait()

        @plsc.parallel_loop(goff, goff + cg, carry=gm)
        def _groups(g, gm_c):
            accs = [buf[pl.ds(g * GELEMS + u * L, L)] for u in range(U)]
            for b in range(U, G):
                v = buf[pl.ds(g * GELEMS + b * L, L)]
                u = b % U
                accs[u] = jnp.minimum(accs[u], v)
            gv = jnp.minimum(
                jnp.minimum(accs[0], accs[1]), jnp.minimum(accs[2], accs[3])
            )
            gmin_ref[pl.ds(g * L, L)] = gv
            return jnp.minimum(gm_c, gv)

        gm = _groups
        goff += cg

    m = gm[0]
    for l in range(1, L):
        v = gm[l]
        m = jnp.where(v < m, v, m)
    msplat = jnp.full((L,), m, jnp.float32)

    cg = jnp.full((L,), NG, jnp.int32)
    for g in range(NG):
        gv = gmin_ref[pl.ds(g * L, L)]
        cg = jnp.minimum(cg, jnp.where(gv == msplat, jnp.int32(g), NG))
    bestg = cg[0]
    for l in range(1, L):
        i = cg[l]
        bestg = jnp.where(i < bestg, i, bestg)

    gbase = bestg * GELEMS
    ci = jnp.full((L,), INT_MAX, jnp.int32)
    iv = base + gbase + lane

    @plsc.parallel_loop(0, G, carry=(ci, iv), unroll=4)
    def _scan(b, carry):
        ci_c, iv_c = carry
        v = buf[pl.ds(gbase + b * L, L)]
        cand = jnp.where(v == msplat, iv_c, INT_MAX)
        return jnp.minimum(ci_c, cand), iv_c + L

    ci, _ = _scan

    bi = ci[0]
    for l in range(1, L):
        i = ci[l]
        bi = jnp.where(i < bi, i, bi)

    vmin_ref[...] = msplat
    vidx_ref[...] = jnp.full((L,), bi, jnp.int32)
    pltpu.sync_copy(vmin_ref, sh_v.at[pl.ds(s * L, L)])
    pltpu.sync_copy(vidx_ref, sh_i.at[pl.ds(s * L, L)])
    plsc.subcore_barrier()

    @pl.when(s == 0)
    def _final_merge():
        pltpu.sync_copy(sh_v, mv_ref)
        pltpu.sync_copy(sh_i, mi_ref)
        fv = mv_ref[pl.ds(0, L)]
        fi = mi_ref[pl.ds(0, L)]
        bv2, bi2 = fv[0], fi[0]
        for w in range(1, NS):
            wv = mv_ref[pl.ds(w * L, L)]
            wi = mi_ref[pl.ds(w * L, L)]
            v, i = wv[0], wi[0]
            upd = (v < bv2) | ((v == bv2) & (i < bi2))
            bv2 = jnp.where(upd, v, bv2)
            bi2 = jnp.where(upd, i, bi2)
        vidx_ref[...] = jnp.full((L,), bi2, jnp.int32)
        pltpu.sync_copy(vidx_ref, out)


def kernel(a):
    idx = _sc_argmin(a)
    return idx[0].astype(jnp.int64)


# speedup vs baseline: 1.0213x; 1.0213x over previous
import functools

import jax
import jax.numpy as jnp
from jax import lax
from jax.experimental import pallas as pl
from jax.experimental.pallas import tpu as pltpu
from jax.experimental.pallas import tpu_sc as plsc

R, C = 64, 8192
NS, L = 16, 16
RPW = R // NS
ELEMS = RPW * C
G = 64
GELEMS = G * L
NG = ELEMS // GELEMS
U = 4
NCHUNK = RPW
GPC = NG // NCHUNK
INT_MAX = 2**31 - 1


def _shuffle(x, lane, sh):
    return lax.gather(
        x,
        (lane ^ sh)[:, None],
        dimension_numbers=lax.GatherDimensionNumbers(
            offset_dims=(), collapsed_slice_dims=(0,), start_index_map=(0,)
        ),
        slice_sizes=(1,),
        unique_indices=True,
        mode=lax.GatherScatterMode.PROMISE_IN_BOUNDS,
    )


def _bfly_min(x, lane):
    for sh in (1, 2, 4, 8):
        x = jnp.minimum(x, _shuffle(x, lane, sh))
    return x


def _bfly_lex_min(v, i, lane):
    for sh in (1, 2, 4, 8):
        v2 = _shuffle(v, lane, sh)
        i2 = _shuffle(i, lane, sh)
        upd = (v2 < v) | ((v2 == v) & (i2 < i))
        v = jnp.where(upd, v2, v)
        i = jnp.where(upd, i2, i)
    return v, i


_sc_mesh = plsc.VectorSubcoreMesh(
    core_axis_name="c", subcore_axis_name="s", num_cores=1
)


@functools.partial(
    pl.kernel,
    out_type=jax.ShapeDtypeStruct((L,), jnp.int32),
    mesh=_sc_mesh,
    scratch_types=[
        pltpu.VMEM((ELEMS,), jnp.float32),
        pltpu.VMEM((NG * L,), jnp.float32),
        pltpu.VMEM((L,), jnp.float32),
        pltpu.VMEM((L,), jnp.int32),
        pltpu.VMEM((NS * L,), jnp.float32),
        pltpu.VMEM((NS * L,), jnp.int32),
        pltpu.VMEM_SHARED((NS * L,), jnp.float32),
        pltpu.VMEM_SHARED((NS * L,), jnp.int32),
        [pltpu.SemaphoreType.DMA] * NCHUNK,
    ],
)
def _sc_argmin(
    a_hbm, out, buf, gmin_ref, vmin_ref, vidx_ref, mv_ref, mi_ref,
    sh_v, sh_i, sems
):
    s = lax.axis_index("s")
    row0 = s * RPW
    base = row0 * C

    copies = [
        pltpu.async_copy(
            a_hbm.at[row0 + k], buf.at[pl.ds(k * C, C)], sems[k]
        )
        for k in range(NCHUNK)
    ]

    lane = lax.iota(jnp.int32, L)
    inf = jnp.float32(jnp.inf)

    gm = jnp.full((L,), inf, jnp.float32)
    for k in range(NCHUNK):
        copies[k].wait()

        @plsc.parallel_loop(k * GPC, (k + 1) * GPC, carry=gm)
        def _groups(g, gm_c):
            accs = [buf[pl.ds(g * GELEMS + u * L, L)] for u in range(U)]
            for b in range(U, G):
                v = buf[pl.ds(g * GELEMS + b * L, L)]
                u = b % U
                accs[u] = jnp.minimum(accs[u], v)
            gv = jnp.minimum(
                jnp.minimum(accs[0], accs[1]), jnp.minimum(accs[2], accs[3])
            )
            gmin_ref[pl.ds(g * L, L)] = gv
            return jnp.minimum(gm_c, gv)

        gm = _groups

    msplat = _bfly_min(gm, lane)

    cg = jnp.full((L,), NG, jnp.int32)
    for g in range(NG):
        gv = gmin_ref[pl.ds(g * L, L)]
        cg = jnp.minimum(cg, jnp.where(gv == msplat, jnp.int32(g), NG))
    bestg = _bfly_min(cg, lane)[0]

    gbase = bestg * GELEMS
    ci0 = jnp.full((L,), INT_MAX, jnp.int32)
    iv0 = base + gbase + lane

    @plsc.parallel_loop(0, G, carry=(ci0, iv0), unroll=4)
    def _scan(b, carry):
        ci_c, iv_c = carry
        v = buf[pl.ds(gbase + b * L, L)]
        cand = jnp.where(v == msplat, iv_c, INT_MAX)
        return jnp.minimum(ci_c, cand), iv_c + L

    ci, _ = _scan
    bidx = _bfly_min(ci, lane)

    vmin_ref[...] = msplat
    vidx_ref[...] = bidx
    pltpu.sync_copy(vmin_ref, sh_v.at[pl.ds(s * L, L)])
    pltpu.sync_copy(vidx_ref, sh_i.at[pl.ds(s * L, L)])
    plsc.subcore_barrier()

    @pl.when(s == 0)
    def _final_merge():
        pltpu.sync_copy(sh_v, mv_ref)
        pltpu.sync_copy(sh_i, mi_ref)
        fv = mv_ref[pl.ds(0, L)]
        fi = mi_ref[pl.ds(0, L)]
        for w in range(1, NS):
            wv = mv_ref[pl.ds(w * L, L)]
            wi = mi_ref[pl.ds(w * L, L)]
            upd = (wv < fv) | ((wv == fv) & (wi < fi))
            fv = jnp.where(upd, wv, fv)
            fi = jnp.where(upd, wi, fi)
        vidx_ref[...] = fi
        pltpu.sync_copy(vidx_ref, out)


def kernel(a):
    idx = _sc_argmin(a)
    return idx[0].astype(jnp.int64)


# speedup vs baseline: 1.0339x; 1.0123x over previous
import functools

import jax
import jax.numpy as jnp
from jax import lax
from jax.experimental import pallas as pl
from jax.experimental.pallas import tpu as pltpu
from jax.experimental.pallas import tpu_sc as plsc

R, C = 64, 8192
NS, L = 16, 16
RPW = R // NS
ELEMS = RPW * C
G = 64
GELEMS = G * L
NG = ELEMS // GELEMS
U = 8
NCHUNK = RPW
GPC = NG // NCHUNK
INT_MAX = 2**31 - 1


def _shuffle(x, lane, sh):
    return lax.gather(
        x,
        (lane ^ sh)[:, None],
        dimension_numbers=lax.GatherDimensionNumbers(
            offset_dims=(), collapsed_slice_dims=(0,), start_index_map=(0,)
        ),
        slice_sizes=(1,),
        unique_indices=True,
        mode=lax.GatherScatterMode.PROMISE_IN_BOUNDS,
    )


def _bfly_min(x, lane):
    for sh in (1, 2, 4, 8):
        x = jnp.minimum(x, _shuffle(x, lane, sh))
    return x


def _bfly_lex_min(v, i, lane):
    for sh in (1, 2, 4, 8):
        v2 = _shuffle(v, lane, sh)
        i2 = _shuffle(i, lane, sh)
        upd = (v2 < v) | ((v2 == v) & (i2 < i))
        v = jnp.where(upd, v2, v)
        i = jnp.where(upd, i2, i)
    return v, i


_sc_mesh = plsc.VectorSubcoreMesh(
    core_axis_name="c", subcore_axis_name="s", num_cores=1
)


@functools.partial(
    pl.kernel,
    out_type=jax.ShapeDtypeStruct((L,), jnp.int32),
    mesh=_sc_mesh,
    scratch_types=[
        pltpu.VMEM((ELEMS,), jnp.float32),
        pltpu.VMEM((NG * L,), jnp.float32),
        pltpu.VMEM((L,), jnp.float32),
        pltpu.VMEM((L,), jnp.int32),
        pltpu.VMEM((NS * L,), jnp.float32),
        pltpu.VMEM((NS * L,), jnp.int32),
        pltpu.VMEM_SHARED((NS * L,), jnp.float32),
        pltpu.VMEM_SHARED((NS * L,), jnp.int32),
        [pltpu.SemaphoreType.DMA] * NCHUNK,
    ],
)
def _sc_argmin(
    a_hbm, out, buf, gmin_ref, vmin_ref, vidx_ref, mv_ref, mi_ref,
    sh_v, sh_i, sems
):
    s = lax.axis_index("s")
    row0 = s * RPW
    base = row0 * C

    copies = [
        pltpu.async_copy(
            a_hbm.at[row0 + k], buf.at[pl.ds(k * C, C)], sems[k]
        )
        for k in range(NCHUNK)
    ]

    lane = lax.iota(jnp.int32, L)
    inf = jnp.float32(jnp.inf)

    gm = jnp.full((L,), inf, jnp.float32)
    for k in range(NCHUNK):
        copies[k].wait()

        @plsc.parallel_loop(k * GPC, (k + 1) * GPC, carry=gm)
        def _groups(g, gm_c):
            accs = [buf[pl.ds(g * GELEMS + u * L, L)] for u in range(U)]
            for b in range(U, G):
                v = buf[pl.ds(g * GELEMS + b * L, L)]
                u = b % U
                accs[u] = jnp.minimum(accs[u], v)
            while len(accs) > 1:
                accs = [jnp.minimum(accs[i], accs[i + 1])
                        for i in range(0, len(accs), 2)]
            gv = accs[0]
            gmin_ref[pl.ds(g * L, L)] = gv
            return jnp.minimum(gm_c, gv)

        gm = _groups

    msplat = _bfly_min(gm, lane)

    cg = jnp.full((L,), NG, jnp.int32)
    for g in range(NG):
        gv = gmin_ref[pl.ds(g * L, L)]
        cg = jnp.minimum(cg, jnp.where(gv == msplat, jnp.int32(g), NG))
    bestg = _bfly_min(cg, lane)[0]

    gbase = bestg * GELEMS
    ci0 = jnp.full((L,), INT_MAX, jnp.int32)
    iv0 = base + gbase + lane

    @plsc.parallel_loop(0, G, carry=(ci0, iv0), unroll=4)
    def _scan(b, carry):
        ci_c, iv_c = carry
        v = buf[pl.ds(gbase + b * L, L)]
        cand = jnp.where(v == msplat, iv_c, INT_MAX)
        return jnp.minimum(ci_c, cand), iv_c + L

    ci, _ = _scan
    bidx = _bfly_min(ci, lane)

    vmin_ref[...] = msplat
    vidx_ref[...] = bidx
    pltpu.sync_copy(vmin_ref, sh_v.at[pl.ds(s * L, L)])
    pltpu.sync_copy(vidx_ref, sh_i.at[pl.ds(s * L, L)])
    plsc.subcore_barrier()

    @pl.when(s == 0)
    def _final_merge():
        pltpu.sync_copy(sh_v, mv_ref)
        pltpu.sync_copy(sh_i, mi_ref)
        fv = mv_ref[pl.ds(0, L)]
        fi = mi_ref[pl.ds(0, L)]
        for w in range(1, NS):
            wv = mv_ref[pl.ds(w * L, L)]
            wi = mi_ref[pl.ds(w * L, L)]
            upd = (wv < fv) | ((wv == fv) & (wi < fi))
            fv = jnp.where(upd, wv, fv)
            fi = jnp.where(upd, wi, fi)
        vidx_ref[...] = fi
        pltpu.sync_copy(vidx_ref, out)


def kernel(a):
    idx = _sc_argmin(a)
    return idx[0].astype(jnp.int64)


# speedup vs baseline: 1.0421x; 1.0080x over previous
import functools

import jax
import jax.numpy as jnp
from jax import lax
from jax.experimental import pallas as pl
from jax.experimental.pallas import tpu as pltpu
from jax.experimental.pallas import tpu_sc as plsc

R, C = 64, 8192
HALF = R // 2
NS, L = 16, 16
RPW = HALF // NS
NBLK = C // L
U = 4
INT_MAX = 2**31 - 1

_sc_mesh = plsc.VectorSubcoreMesh(
    core_axis_name="c", subcore_axis_name="s", num_cores=1
)


@functools.partial(
    pl.kernel,
    out_type=[
        jax.ShapeDtypeStruct((NS, L), jnp.float32),
        jax.ShapeDtypeStruct((NS, L), jnp.int32),
    ],
    mesh=_sc_mesh,
    scratch_types=[
        pltpu.VMEM((RPW * C,), jnp.float32),
        pltpu.VMEM((L,), jnp.float32),
        pltpu.VMEM((L,), jnp.int32),
        [pltpu.SemaphoreType.DMA] * RPW,
    ],
)
def _sc_half_argmin(a_hbm, vals_out, idxs_out, buf, vmin_ref, vidx_ref, sems):
    s = lax.axis_index("s")
    row0 = s * RPW
    base = row0 * C

    copies = [
        pltpu.async_copy(a_hbm.at[row0 + k], buf.at[pl.ds(k * C, C)], sems[k])
        for k in range(RPW)
    ]

    lane = lax.iota(jnp.int32, L)
    inf = jnp.float32(jnp.inf)
    vmins = [jnp.full((L,), inf, jnp.float32) for _ in range(U)]
    vidxs = [jnp.zeros((L,), jnp.int32) for _ in range(U)]

    for k in range(RPW):
        copies[k].wait()
        cbase = k * C

        init = tuple(vmins) + tuple(
            base + cbase + u * L + lane for u in range(U)
        ) + tuple(vidxs)

        @plsc.parallel_loop(0, NBLK // U, carry=init, unroll=2)
        def body(i, carry):
            vm = list(carry[:U])
            cur = list(carry[U : 2 * U])
            vi = list(carry[2 * U :])
            for u in range(U):
                v = buf[pl.ds(cbase + (i * U + u) * L, L)]
                upd = v < vm[u]
                vm[u] = jnp.where(upd, v, vm[u])
                vi[u] = jnp.where(upd, cur[u], vi[u])
                cur[u] = cur[u] + U * L
            return tuple(vm) + tuple(cur) + tuple(vi)

        out_carry = body
        vmins = list(out_carry[:U])
        vidxs = list(out_carry[2 * U :])

    vmin, vidx = vmins[0], vidxs[0]
    for u in range(1, U):
        upd = (vmins[u] < vmin) | ((vmins[u] == vmin) & (vidxs[u] < vidx))
        vmin = jnp.where(upd, vmins[u], vmin)
        vidx = jnp.where(upd, vidxs[u], vidx)

    vmin_ref[...] = vmin
    vidx_ref[...] = vidx
    pltpu.sync_copy(vmin_ref, vals_out.at[s])
    pltpu.sync_copy(vidx_ref, idxs_out.at[s])


def _tc_scan_body(a_ref, val_ref, idx_ref):
    v = a_ref[...]
    m = jnp.min(v)
    row = lax.broadcasted_iota(jnp.int32, (HALF, C), 0)
    col = lax.broadcasted_iota(jnp.int32, (HALF, C), 1)
    flat = (row + HALF) * C + col
    cand = jnp.where(v == m, flat, INT_MAX)
    val_ref[0, 0] = m
    idx_ref[0, 0] = jnp.min(cand)


_tc_scan = pl.pallas_call(
    _tc_scan_body,
    grid=(1,),
    in_specs=[pl.BlockSpec((HALF, C), lambda i: (1, 0))],
    out_shape=[
        jax.ShapeDtypeStruct((1, 1), jnp.float32),
        jax.ShapeDtypeStruct((1, 1), jnp.int32),
    ],
    out_specs=[
        pl.BlockSpec(memory_space=pltpu.SMEM),
        pl.BlockSpec(memory_space=pltpu.SMEM),
    ],
)


def _merge_body(vals_ref, idxs_ref, tcv_ref, tci_ref, out_ref):
    vals = vals_ref[...]
    idxs = idxs_ref[...]
    tcv = tcv_ref[0, 0]
    tci = tci_ref[0, 0]
    m = jnp.minimum(jnp.min(vals), tcv)
    sc_best = jnp.min(jnp.where(vals == m, idxs, INT_MAX))
    tc_best = jnp.where(tcv == m, tci, INT_MAX)
    out_ref[0, 0] = jnp.minimum(sc_best, tc_best)


_merge = pl.pallas_call(
    _merge_body,
    in_specs=[
        pl.BlockSpec((NS, L), lambda: (0, 0)),
        pl.BlockSpec((NS, L), lambda: (0, 0)),
        pl.BlockSpec(memory_space=pltpu.SMEM),
        pl.BlockSpec(memory_space=pltpu.SMEM),
    ],
    out_shape=jax.ShapeDtypeStruct((1, 1), jnp.int32),
    out_specs=pl.BlockSpec(memory_space=pltpu.SMEM),
)


def kernel(a):
    vals, idxs = _sc_half_argmin(a)
    tcv, tci = _tc_scan(a)
    out = _merge(vals, idxs, tcv, tci)
    return out[0, 0].astype(jnp.int64)


# speedup vs baseline: 1.1346x; 1.0887x over previous
import functools

import jax
import jax.numpy as jnp
from jax import lax
from jax.experimental import pallas as pl
from jax.experimental.pallas import tpu as pltpu
from jax.experimental.pallas import tpu_sc as plsc

R, C = 64, 8192
NS, L = 16, 16
SC_ROWS = 16
TC_ROWS = R - SC_ROWS
TB = 8
NTB = TC_ROWS // TB
NCHUNK = 2
CHUNK = C // NCHUNK
NBLK = CHUNK // L
U = 4
INT_MAX = 2**31 - 1

_sc_mesh = plsc.VectorSubcoreMesh(
    core_axis_name="c", subcore_axis_name="s", num_cores=1
)


@functools.partial(
    pl.kernel,
    out_type=[
        jax.ShapeDtypeStruct((NS, L), jnp.float32),
        jax.ShapeDtypeStruct((NS, L), jnp.int32),
    ],
    mesh=_sc_mesh,
    scratch_types=[
        pltpu.VMEM((C,), jnp.float32),
        pltpu.VMEM((L,), jnp.float32),
        pltpu.VMEM((L,), jnp.int32),
        [pltpu.SemaphoreType.DMA] * NCHUNK,
    ],
)
def _sc_part_argmin(a_hbm, vals_out, idxs_out, buf, vmin_ref, vidx_ref, sems):
    s = lax.axis_index("s")
    base = s * C

    copies = [
        pltpu.async_copy(
            a_hbm.at[s, pl.ds(k * CHUNK, CHUNK)],
            buf.at[pl.ds(k * CHUNK, CHUNK)],
            sems[k],
        )
        for k in range(NCHUNK)
    ]

    lane = lax.iota(jnp.int32, L)
    inf = jnp.float32(jnp.inf)
    vmins = [jnp.full((L,), inf, jnp.float32) for _ in range(U)]
    vidxs = [jnp.zeros((L,), jnp.int32) for _ in range(U)]

    for k in range(NCHUNK):
        copies[k].wait()
        cbase = k * CHUNK

        init = tuple(vmins) + tuple(
            base + cbase + u * L + lane for u in range(U)
        ) + tuple(vidxs)

        @plsc.parallel_loop(0, NBLK // U, carry=init, unroll=2)
        def body(i, carry):
            vm = list(carry[:U])
            cur = list(carry[U : 2 * U])
            vi = list(carry[2 * U :])
            for u in range(U):
                v = buf[pl.ds(cbase + (i * U + u) * L, L)]
                upd = v < vm[u]
                vm[u] = jnp.where(upd, v, vm[u])
                vi[u] = jnp.where(upd, cur[u], vi[u])
                cur[u] = cur[u] + U * L
            return tuple(vm) + tuple(cur) + tuple(vi)

        out_carry = body
        vmins = list(out_carry[:U])
        vidxs = list(out_carry[2 * U :])

    vmin, vidx = vmins[0], vidxs[0]
    for u in range(1, U):
        upd = (vmins[u] < vmin) | ((vmins[u] == vmin) & (vidxs[u] < vidx))
        vmin = jnp.where(upd, vmins[u], vmin)
        vidx = jnp.where(upd, vidxs[u], vidx)

    vmin_ref[...] = vmin
    vidx_ref[...] = vidx
    pltpu.sync_copy(vmin_ref, vals_out.at[s])
    pltpu.sync_copy(vidx_ref, idxs_out.at[s])


def _tc_scan_body(a_ref, val_ref, idx_ref, mcar, icar):
    i = pl.program_id(0)
    v = a_ref[...]
    m = jnp.min(v)
    row = lax.broadcasted_iota(jnp.int32, (TB, C), 0)
    col = lax.broadcasted_iota(jnp.int32, (TB, C), 1)
    flat = (row + SC_ROWS + i * TB) * C + col
    mi = jnp.min(jnp.where(v == m, flat, INT_MAX))

    @pl.when(i == 0)
    def _():
        mcar[0] = jnp.float32(jnp.inf)
        icar[0] = jnp.int32(INT_MAX)

    upd = m < mcar[0]
    mcar[0] = jnp.where(upd, m, mcar[0])
    icar[0] = jnp.where(upd, mi, icar[0])

    @pl.when(i == NTB - 1)
    def _():
        val_ref[...] = jnp.full((1, 128), mcar[0], jnp.float32)
        idx_ref[...] = jnp.full((1, 128), icar[0], jnp.int32)


_tc_scan = pl.pallas_call(
    _tc_scan_body,
    grid=(NTB,),
    in_specs=[pl.BlockSpec((TB, C), lambda i: (i + SC_ROWS // TB, 0))],
    out_shape=[
        jax.ShapeDtypeStruct((1, 128), jnp.float32),
        jax.ShapeDtypeStruct((1, 128), jnp.int32),
    ],
    out_specs=[
        pl.BlockSpec((1, 128), lambda i: (0, 0)),
        pl.BlockSpec((1, 128), lambda i: (0, 0)),
    ],
    scratch_shapes=[
        pltpu.SMEM((1,), jnp.float32),
        pltpu.SMEM((1,), jnp.int32),
    ],
)


def _merge_body(vals_ref, idxs_ref, tcv_ref, tci_ref, out_ref):
    vals = vals_ref[...]
    idxs = idxs_ref[...]
    tcv = tcv_ref[...]
    tci = tci_ref[...]
    m = jnp.minimum(jnp.min(vals), jnp.min(tcv))
    sc_best = jnp.min(jnp.where(vals == m, idxs, INT_MAX))
    tc_best = jnp.min(jnp.where(tcv == m, tci, INT_MAX))
    out_ref[0, 0] = jnp.minimum(sc_best, tc_best)


_merge = pl.pallas_call(
    _merge_body,
    out_shape=jax.ShapeDtypeStruct((1, 1), jnp.int32),
    out_specs=pl.BlockSpec(memory_space=pltpu.SMEM),
)


def kernel(a):
    vals, idxs = _sc_part_argmin(a)
    tcv, tci = _tc_scan(a)
    out = _merge(vals, idxs, tcv, tci)
    return out[0, 0].astype(jnp.int64)
